# baseline (speedup 1.0000x reference)
import math

import jax
import jax.numpy as jnp
from jax import lax
from jax.experimental import pallas as pl
from jax.experimental.pallas import tpu as pltpu
from jax.experimental.pallas import tpu_sc as plsc

NUM_BIN = 12
B, MAX_OBJS, H, W = 16, 128, 152, 152
C = 2 * NUM_BIN
HW = H * W
NOBJ = B * MAX_OBJS
C_HALF = C // 2


def _gather_body(feat_hbm, h_hbm, w_hbm, pred_hbm, hv, wv, buf_a, buf_b,
                 out_v, sem_a, sem_b):
    wid = lax.axis_index("s") * 2 + lax.axis_index("c")
    b = lax.shift_right_logical(wid, 1)
    half = jnp.bitwise_and(wid, 1)
    base_o = b * MAX_OBJS

    pltpu.sync_copy(h_hbm.at[pl.ds(base_o, MAX_OBJS)], hv)
    pltpu.sync_copy(w_hbm.at[pl.ds(base_o, MAX_OBJS)], wv)

    bufs = (buf_a, buf_b)
    sems = (sem_a, sem_b)

    def fire(k):
        return pltpu.async_copy(feat_hbm.at[b, half * C_HALF + k],
                                bufs[k % 2], sems[k % 2])

    def extract(k, cp):
        cp.wait()
        buf = bufs[k % 2]
        for q in range(MAX_OBJS // 16):
            hs = hv[pl.ds(q * 16, 16)]
            ws = wv[pl.ds(q * 16, 16)]
            out_v[k, pl.ds(q * 16, 16)] = plsc.load_gather(buf, [hs, ws])

    pending = fire(0)
    for k in range(C_HALF):
        nxt = fire(k + 1) if k + 1 < C_HALF else None
        extract(k, pending)
        pending = nxt

    pltpu.sync_copy(out_v, pred_hbm.at[half, :, pl.ds(base_o, MAX_OBJS)])


def _sc_gather(feat, h_idx, w_idx):
    mesh = plsc.VectorSubcoreMesh(core_axis_name="c", subcore_axis_name="s")
    return pl.kernel(
        _gather_body,
        out_type=jax.ShapeDtypeStruct((2, C_HALF, NOBJ), jnp.float32),
        mesh=mesh,
        scratch_types=[
            pltpu.VMEM((MAX_OBJS,), jnp.int32),
            pltpu.VMEM((MAX_OBJS,), jnp.int32),
            pltpu.VMEM((H, W), jnp.float32),
            pltpu.VMEM((H, W), jnp.float32),
            pltpu.VMEM((C_HALF, MAX_OBJS), jnp.float32),
            pltpu.SemaphoreType.DMA,
            pltpu.SemaphoreType.DMA,
        ],
        compiler_params=pltpu.CompilerParams(needs_layout_passes=False),
    )(feat, h_idx, w_idx)


def _loss_body(pred_ref, maskf_ref, targ_ref, out_ref):
    two_pi = 2.0 * math.pi
    apc = two_pi / NUM_BIN

    pred = pred_ref[...]
    m = maskf_ref[...]
    ry = targ_ref[...]

    heading = jnp.mod(ry, two_pi)
    shift = jnp.mod(heading + apc / 2.0, two_pi)
    binf = jnp.floor(shift / apc)
    bin_i = binf.astype(jnp.int32)
    res_norm = (shift - (binf * apc + apc / 2.0)) / (apc / 2.0)

    logits = pred[:NUM_BIN, :]
    mx = jnp.max(logits, axis=0, keepdims=True)
    sh = logits - mx
    lse = jnp.log(jnp.sum(jnp.exp(sh), axis=0, keepdims=True))
    logp = sh - lse

    iota = lax.broadcasted_iota(jnp.int32, (NUM_BIN, NOBJ), 0)
    onehot = (iota == bin_i).astype(jnp.float32)

    per_bin = jnp.sum(logp * onehot, axis=0, keepdims=True) * m
    res_pred = jnp.sum(pred[NUM_BIN:, :] * onehot, axis=0, keepdims=True)
    diff = res_pred - res_norm
    ad = jnp.abs(diff)
    per_res = jnp.where(ad < 1.0, 0.5 * diff * diff, ad - 0.5) * m

    denom = jnp.maximum(jnp.sum(m), 1.0)
    out_ref[0, 0] = (jnp.sum(per_res) - jnp.sum(per_bin)) / denom


def _tc_loss(pred_t, maskf, targ):
    out = pl.pallas_call(
        _loss_body,
        out_shape=jax.ShapeDtypeStruct((1, 1), jnp.float32),
        out_specs=pl.BlockSpec(memory_space=pltpu.SMEM),
    )(pred_t, maskf, targ)
    return out.reshape(())


def kernel(output, mask, ind, target):
    ind_flat = ind.reshape(NOBJ)
    h_idx = ind_flat // W
    w_idx = ind_flat - h_idx * W
    pred_t = _sc_gather(output, h_idx, w_idx).reshape(C, NOBJ)
    maskf = mask.reshape(1, NOBJ).astype(jnp.float32)
    targ = target.reshape(1, NOBJ)
    return _tc_loss(pred_t, maskf, targ)


# speedup vs baseline: 1.0427x; 1.0427x over previous
import math

import jax
import jax.numpy as jnp
from jax import lax
from jax.experimental import pallas as pl
from jax.experimental.pallas import tpu as pltpu
from jax.experimental.pallas import tpu_sc as plsc

NUM_BIN = 12
B, MAX_OBJS, H, W = 16, 128, 152, 152
C = 2 * NUM_BIN
HW = H * W
NOBJ = B * MAX_OBJS
C_HALF = C // 2


NBUF = 3


def _gather_body(feat_hbm, h_hbm, w_hbm, pred_hbm, hv, wv, buf_a, buf_b,
                 buf_c, out_v, sem_a, sem_b, sem_c):
    wid = lax.axis_index("s") * 2 + lax.axis_index("c")
    b = lax.shift_right_logical(wid, 1)
    half = jnp.bitwise_and(wid, 1)
    base_o = b * MAX_OBJS

    pltpu.sync_copy(h_hbm.at[pl.ds(base_o, MAX_OBJS)], hv)
    pltpu.sync_copy(w_hbm.at[pl.ds(base_o, MAX_OBJS)], wv)

    bufs = (buf_a, buf_b, buf_c)
    sems = (sem_a, sem_b, sem_c)

    def fire(k):
        return pltpu.async_copy(feat_hbm.at[b, half * C_HALF + k],
                                bufs[k % NBUF], sems[k % NBUF])

    def extract(k, cp):
        cp.wait()
        buf = bufs[k % NBUF]
        for q in range(MAX_OBJS // 16):
            hs = hv[pl.ds(q * 16, 16)]
            ws = wv[pl.ds(q * 16, 16)]
            out_v[k, pl.ds(q * 16, 16)] = plsc.load_gather(buf, [hs, ws])

    pending = [fire(k) for k in range(NBUF - 1)]
    for k in range(C_HALF):
        if k + NBUF - 1 < C_HALF:
            pending.append(fire(k + NBUF - 1))
        extract(k, pending.pop(0))

    pltpu.sync_copy(out_v, pred_hbm.at[half, :, pl.ds(base_o, MAX_OBJS)])


def _sc_gather(feat, h_idx, w_idx):
    mesh = plsc.VectorSubcoreMesh(core_axis_name="c", subcore_axis_name="s")
    return pl.kernel(
        _gather_body,
        out_type=jax.ShapeDtypeStruct((2, C_HALF, NOBJ), jnp.float32),
        mesh=mesh,
        scratch_types=[
            pltpu.VMEM((MAX_OBJS,), jnp.int32),
            pltpu.VMEM((MAX_OBJS,), jnp.int32),
            pltpu.VMEM((H, W), jnp.float32),
            pltpu.VMEM((H, W), jnp.float32),
            pltpu.VMEM((H, W), jnp.float32),
            pltpu.VMEM((C_HALF, MAX_OBJS), jnp.float32),
            pltpu.SemaphoreType.DMA,
            pltpu.SemaphoreType.DMA,
            pltpu.SemaphoreType.DMA,
        ],
        compiler_params=pltpu.CompilerParams(needs_layout_passes=False),
    )(feat, h_idx, w_idx)


def _loss_body(pred_ref, maskf_ref, targ_ref, out_ref):
    two_pi = 2.0 * math.pi
    apc = two_pi / NUM_BIN

    logits = pred_ref[0]
    res = pred_ref[1]
    m = maskf_ref[...]
    ry = targ_ref[...]

    heading = jnp.mod(ry, two_pi)
    shift = jnp.mod(heading + apc / 2.0, two_pi)
    binf = jnp.floor(shift / apc)
    bin_i = binf.astype(jnp.int32)
    res_norm = (shift - (binf * apc + apc / 2.0)) / (apc / 2.0)

    mx = jnp.max(logits, axis=0, keepdims=True)
    sh = logits - mx
    lse = jnp.log(jnp.sum(jnp.exp(sh), axis=0, keepdims=True))
    logp = sh - lse

    iota = lax.broadcasted_iota(jnp.int32, (NUM_BIN, NOBJ), 0)
    onehot = (iota == bin_i).astype(jnp.float32)

    per_bin = jnp.sum(logp * onehot, axis=0, keepdims=True) * m
    res_pred = jnp.sum(res * onehot, axis=0, keepdims=True)
    diff = res_pred - res_norm
    ad = jnp.abs(diff)
    per_res = jnp.where(ad < 1.0, 0.5 * diff * diff, ad - 0.5) * m

    denom = jnp.maximum(jnp.sum(m), 1.0)
    out_ref[0, 0] = (jnp.sum(per_res) - jnp.sum(per_bin)) / denom


def _tc_loss(pred_t, maskf, targ):
    out = pl.pallas_call(
        _loss_body,
        out_shape=jax.ShapeDtypeStruct((1, 1), jnp.float32),
        out_specs=pl.BlockSpec(memory_space=pltpu.SMEM),
    )(pred_t, maskf, targ)
    return out.reshape(())


def kernel(output, mask, ind, target):
    ind_flat = ind.reshape(NOBJ)
    h_idx = ind_flat // W
    w_idx = ind_flat - h_idx * W
    pred3 = _sc_gather(output, h_idx, w_idx)
    maskf = mask.reshape(1, NOBJ).astype(jnp.float32)
    targ = target.reshape(1, NOBJ)
    return _tc_loss(pred3, maskf, targ)
